# baseline (speedup 1.0000x reference)
import jax
import jax.numpy as jnp
from jax import lax
from jax.experimental import pallas as pl
from jax.experimental.pallas import tpu as pltpu
from jax.experimental.pallas import tpu_sc as plsc

N = 10000
E = 320000
H = 128
NP = 10240
NC = 2
NS = 16
NW = NC * NS
EPT = NP
EP = NW * EPT
K = 128
CH = EPT // K
KA = 80
CHA = EPT // KA
RPT = NP // NS

_mesh = plsc.VectorSubcoreMesh(core_axis_name="c", subcore_axis_name="s")
_f32 = jnp.float32


def _deg_body(srcr, dstr, z1, degout, degin,
              srcb, dstb, ones_v, go_sp, gi_sp):
    c = lax.axis_index("c")
    s = lax.axis_index("s")
    wid = c * NS + s
    pltpu.sync_copy(z1.at[pl.ds(s * RPT, RPT)], go_sp.at[pl.ds(s * RPT, RPT)])
    pltpu.sync_copy(z1.at[pl.ds(s * RPT, RPT)], gi_sp.at[pl.ds(s * RPT, RPT)])
    pltpu.sync_copy(srcr.at[wid], srcb)
    pltpu.sync_copy(dstr.at[wid], dstb)
    for off in range(0, KA - 15, 16):
        ones_v[pl.ds(off, 16)] = jnp.ones((16,), _f32)
    plsc.subcore_barrier()

    def chunk(j, carry):
        pltpu.sync_copy(ones_v, go_sp.at[srcb.at[j]], add=True)
        pltpu.sync_copy(ones_v, gi_sp.at[dstb.at[j]], add=True)
        return carry

    lax.fori_loop(0, CHA, chunk, 0, unroll=False)
    plsc.subcore_barrier()

    @pl.when(s == 0)
    def _():
        pltpu.sync_copy(go_sp, degout.at[c])
        pltpu.sync_copy(gi_sp, degin.at[c])


def _deg_call(srcr, dstr, z1):
    return pl.kernel(
        _deg_body,
        out_type=(
            jax.ShapeDtypeStruct((NC, NP), _f32),
            jax.ShapeDtypeStruct((NC, NP), _f32),
        ),
        mesh=_mesh,
        scratch_types=dict(
            srcb=pltpu.VMEM((CHA, KA), jnp.int32),
            dstb=pltpu.VMEM((CHA, KA), jnp.int32),
            ones_v=pltpu.VMEM((KA,), _f32),
            go_sp=pltpu.VMEM_SHARED((NP,), _f32),
            gi_sp=pltpu.VMEM_SHARED((NP,), _f32),
        ),
    )(srcr, dstr, z1)


def _norm_mm_body(x_ref, w1_ref, dgo_ref, dgi_ref,
                  hsc_ref, nsrc_ref, ndst_ref):
    dgo = dgo_ref[:, 0:1] + dgo_ref[:, 1:2]
    dgi = dgi_ref[:, 0:1] + dgi_ref[:, 1:2]
    nsrc = lax.rsqrt(jnp.maximum(dgo, 1.0))
    ndst = lax.rsqrt(jnp.maximum(dgi, 1.0))
    nsrc_ref[...] = nsrc
    ndst_ref[...] = ndst
    xw = jnp.dot(x_ref[...], w1_ref[...], preferred_element_type=_f32)
    hsc_ref[...] = xw * nsrc


def _norm_mm_call(x_pad, w1, dgo_t, dgi_t):
    return pl.pallas_call(
        _norm_mm_body,
        out_shape=(
            jax.ShapeDtypeStruct((NP, H), _f32),
            jax.ShapeDtypeStruct((NP, 1), _f32),
            jax.ShapeDtypeStruct((NP, 1), _f32),
        ),
    )(x_pad, w1, dgo_t, dgi_t)


def _agg_body(srcf, dstf, hsc, ndst1, z1, z2, agg, wout,
              sidx, didx, stage, wvals, acc_sp, w_sp, isem, gsem, wsem):
    c = lax.axis_index("c")
    s = lax.axis_index("s")
    wid = c * NS + s
    base = wid * EPT
    pltpu.sync_copy(z2.at[pl.ds(s * RPT, RPT)], acc_sp.at[pl.ds(s * RPT, RPT)])
    pltpu.sync_copy(z1.at[pl.ds(s * RPT, RPT)], w_sp.at[pl.ds(s * RPT, RPT)])
    plsc.subcore_barrier()

    def fetch_idx(j, slot):
        pltpu.async_copy(srcf.at[pl.ds(base + j * K, K)], sidx.at[slot], isem)
        pltpu.async_copy(dstf.at[pl.ds(base + j * K, K)], didx.at[slot], isem)

    def wait_idx(j, slot):
        pltpu.make_async_copy(
            srcf.at[pl.ds(base + j * K, K)], sidx.at[slot], isem).wait()
        pltpu.make_async_copy(
            dstf.at[pl.ds(base + j * K, K)], didx.at[slot], isem).wait()

    def issue_gathers(j, slot, b):
        pltpu.async_copy(hsc.at[sidx.at[slot]], stage.at[b], gsem)
        pltpu.async_copy(ndst1.at[didx.at[slot]], wvals.at[b], wsem)

    for j in range(4):
        fetch_idx(j, j)
    for j in range(2):
        wait_idx(j, j)
        issue_gathers(j, j, j)

    def chunk(j, carry):
        b2 = lax.rem(j, 2)
        b4 = lax.rem(j, 4)
        pltpu.make_async_copy(hsc.at[sidx.at[b4]], stage.at[b2], gsem).wait()
        pltpu.make_async_copy(ndst1.at[didx.at[b4]], wvals.at[b2], wsem).wait()
        pltpu.sync_copy(stage.at[b2], acc_sp.at[didx.at[b4]], add=True)
        pltpu.sync_copy(wvals.at[b2], w_sp.at[sidx.at[b4]], add=True)

        @pl.when(j + 4 < CH)
        def _():
            fetch_idx(j + 4, b4)

        @pl.when(j + 2 < CH)
        def _():
            wait_idx(j + 2, lax.rem(j + 2, 4))
            issue_gathers(j + 2, lax.rem(j + 2, 4), b2)

        return carry

    lax.fori_loop(0, CH, chunk, 0, unroll=False)
    plsc.subcore_barrier()

    pltpu.sync_copy(acc_sp.at[pl.ds(s * RPT, RPT)],
                    agg.at[c, pl.ds(s * RPT, RPT)])

    @pl.when(s == 0)
    def _():
        pltpu.sync_copy(w_sp, wout.at[c])


def _agg_call(srcf, dstf, hsc, ndst1, z1, z2):
    return pl.kernel(
        _agg_body,
        out_type=(
            jax.ShapeDtypeStruct((NC, NP, H), _f32),
            jax.ShapeDtypeStruct((NC, NP), _f32),
        ),
        mesh=_mesh,
        scratch_types=dict(
            sidx=pltpu.VMEM((4, K), jnp.int32),
            didx=pltpu.VMEM((4, K), jnp.int32),
            stage=pltpu.VMEM((2, K, H), _f32),
            wvals=pltpu.VMEM((2, K), _f32),
            acc_sp=pltpu.VMEM_SHARED((NP, H), _f32),
            w_sp=pltpu.VMEM_SHARED((NP,), _f32),
            isem=pltpu.SemaphoreType.DMA,
            gsem=pltpu.SemaphoreType.DMA,
            wsem=pltpu.SemaphoreType.DMA,
        ),
    )(srcf, dstf, hsc, ndst1, z1, z2)


def _final_body(agg_ref, ndst_ref, nsrc_ref, wp_ref,
                b1_ref, w2_ref, b2_ref, wc_ref, bc_ref, out_ref):
    agg = agg_ref[0] + agg_ref[1]
    h1 = jnp.maximum(agg * ndst_ref[...] + b1_ref[...], 0.0)
    wsum = wp_ref[:, 0:1] + wp_ref[:, 1:2]
    real = (lax.broadcasted_iota(jnp.int32, (NP, 1), 0) < N).astype(_f32)
    coeff = wsum * nsrc_ref[...] * real
    u = lax.dot_general(coeff, h1, (((0,), (0,)), ((), ())),
                        preferred_element_type=_f32)
    hg = jnp.dot(u, w2_ref[...], preferred_element_type=_f32) * (1.0 / N)
    hg = hg + b2_ref[...]
    out_ref[...] = jnp.dot(hg, wc_ref[...], preferred_element_type=_f32) \
        + bc_ref[...]


def _final_call(agg, ndst, nsrc, wp_t, b1, w2, b2, wc, bc):
    return pl.pallas_call(
        _final_body,
        out_shape=jax.ShapeDtypeStruct((1, 10), _f32),
    )(agg, ndst, nsrc, wp_t, b1, w2, b2, wc, bc)


@jax.jit
def kernel(in_feat, edge_index, W1, b1, W2, b2, Wc, bc):
    pad = jnp.full((EP - E,), NP - 1, jnp.int32)
    srcf = jnp.concatenate([edge_index[0], pad])
    dstf = jnp.concatenate([edge_index[1], pad])
    x_pad = jnp.concatenate([in_feat, jnp.zeros((NP - N, H), _f32)])
    z1 = jnp.zeros((NP,), _f32)
    z2 = jnp.zeros((NP, H), _f32)

    degout, degin = _deg_call(srcf.reshape(NW, CHA, KA),
                              dstf.reshape(NW, CHA, KA), z1)
    hsc, nsrc, ndst = _norm_mm_call(x_pad, W1, degout.T, degin.T)
    agg, w_parts = _agg_call(srcf, dstf, hsc, ndst[:, 0], z1, z2)
    return _final_call(agg, ndst, nsrc, w_parts.T,
                       b1.reshape(1, H), W2, b2.reshape(1, H),
                       Wc.reshape(H, 10), bc.reshape(1, 10))


# speedup vs baseline: 2.3550x; 2.3550x over previous
import jax
import jax.numpy as jnp
from jax import lax
from jax.experimental import pallas as pl
from jax.experimental.pallas import tpu as pltpu
from jax.experimental.pallas import tpu_sc as plsc

N = 10000
E = 320000
H = 128
NP = 10240
NC = 2
NS = 16
NW = NC * NS
EPT = NP
EP = NW * EPT
K = 128
CH = EPT // K
KA = 80
CHA = EPT // KA
RPT = NP // NS

_mesh = plsc.VectorSubcoreMesh(core_axis_name="c", subcore_axis_name="s")
_f32 = jnp.float32


def _deg_body(srcr, dstr, z1, degout, degin,
              srcb, dstb, ones_v, go_sp, gi_sp):
    c = lax.axis_index("c")
    s = lax.axis_index("s")
    wid = c * NS + s
    pltpu.sync_copy(z1.at[pl.ds(s * RPT, RPT)], go_sp.at[pl.ds(s * RPT, RPT)])
    pltpu.sync_copy(z1.at[pl.ds(s * RPT, RPT)], gi_sp.at[pl.ds(s * RPT, RPT)])
    pltpu.sync_copy(srcr.at[wid], srcb)
    pltpu.sync_copy(dstr.at[wid], dstb)
    for off in range(0, KA - 15, 16):
        ones_v[pl.ds(off, 16)] = jnp.ones((16,), _f32)
    plsc.subcore_barrier()

    def chunk(j, carry):
        pltpu.sync_copy(ones_v, go_sp.at[srcb.at[j]], add=True)
        pltpu.sync_copy(ones_v, gi_sp.at[dstb.at[j]], add=True)
        return carry

    lax.fori_loop(0, CHA, chunk, 0, unroll=False)
    plsc.subcore_barrier()

    @pl.when(s == 0)
    def _():
        pltpu.sync_copy(go_sp, degout.at[c])
        pltpu.sync_copy(gi_sp, degin.at[c])


def _deg_call(srcr, dstr, z1):
    return pl.kernel(
        _deg_body,
        out_type=(
            jax.ShapeDtypeStruct((NC, NP), _f32),
            jax.ShapeDtypeStruct((NC, NP), _f32),
        ),
        mesh=_mesh,
        scratch_types=dict(
            srcb=pltpu.VMEM((CHA, KA), jnp.int32),
            dstb=pltpu.VMEM((CHA, KA), jnp.int32),
            ones_v=pltpu.VMEM((KA,), _f32),
            go_sp=pltpu.VMEM_SHARED((NP,), _f32),
            gi_sp=pltpu.VMEM_SHARED((NP,), _f32),
        ),
    )(srcr, dstr, z1)


def _norm_mm_body(x_ref, w1_ref, dgo_ref, dgi_ref,
                  hsc_ref, nsrc_ref, ndst_ref):
    dgo = dgo_ref[:, 0:1] + dgo_ref[:, 1:2]
    dgi = dgi_ref[:, 0:1] + dgi_ref[:, 1:2]
    nsrc = lax.rsqrt(jnp.maximum(dgo, 1.0))
    ndst = lax.rsqrt(jnp.maximum(dgi, 1.0))
    nsrc_ref[...] = nsrc
    ndst_ref[...] = ndst
    xw = jnp.dot(x_ref[...], w1_ref[...], preferred_element_type=_f32)
    hsc_ref[...] = xw * nsrc


def _norm_mm_call(x_pad, w1, dgo_t, dgi_t):
    return pl.pallas_call(
        _norm_mm_body,
        out_shape=(
            jax.ShapeDtypeStruct((NP, H), _f32),
            jax.ShapeDtypeStruct((NP, 1), _f32),
            jax.ShapeDtypeStruct((NP, 1), _f32),
        ),
    )(x_pad, w1, dgo_t, dgi_t)


def _agg_body(srcf, dstf, hsc, ndst1, z1, z2, agg, wout,
              sidx, didx, stage, wvals, acc_sp, w_sp, isem, gsem, wsem):
    c = lax.axis_index("c")
    s = lax.axis_index("s")
    wid = c * NS + s
    base = wid * EPT
    pltpu.sync_copy(z2.at[pl.ds(s * RPT, RPT)], acc_sp.at[pl.ds(s * RPT, RPT)])
    pltpu.sync_copy(z1.at[pl.ds(s * RPT, RPT)], w_sp.at[pl.ds(s * RPT, RPT)])
    plsc.subcore_barrier()

    def fetch_idx(j, slot):
        pltpu.async_copy(srcf.at[pl.ds(base + j * K, K)], sidx.at[slot], isem)
        pltpu.async_copy(dstf.at[pl.ds(base + j * K, K)], didx.at[slot], isem)

    def wait_idx(j, slot):
        pltpu.make_async_copy(
            srcf.at[pl.ds(base + j * K, K)], sidx.at[slot], isem).wait()
        pltpu.make_async_copy(
            dstf.at[pl.ds(base + j * K, K)], didx.at[slot], isem).wait()

    def issue_gathers(j, slot, b):
        pltpu.async_copy(hsc.at[sidx.at[slot]], stage.at[b], gsem)
        pltpu.async_copy(ndst1.at[didx.at[slot]], wvals.at[b], wsem)

    for j in range(4):
        fetch_idx(j, j)
    for j in range(2):
        wait_idx(j, j)
        issue_gathers(j, j, j)

    def chunk(j, carry):
        b2 = lax.rem(j, 2)
        b4 = lax.rem(j, 4)
        pltpu.make_async_copy(hsc.at[sidx.at[b4]], stage.at[b2], gsem).wait()
        pltpu.make_async_copy(ndst1.at[didx.at[b4]], wvals.at[b2], wsem).wait()
        pltpu.sync_copy(stage.at[b2], acc_sp.at[didx.at[b4]], add=True)
        pltpu.sync_copy(wvals.at[b2], w_sp.at[sidx.at[b4]], add=True)

        @pl.when(j + 4 < CH)
        def _():
            fetch_idx(j + 4, b4)

        @pl.when(j + 2 < CH)
        def _():
            wait_idx(j + 2, lax.rem(j + 2, 4))
            issue_gathers(j + 2, lax.rem(j + 2, 4), b2)

        return carry

    lax.fori_loop(0, CH, chunk, 0, unroll=False)
    plsc.subcore_barrier()

    pltpu.sync_copy(acc_sp.at[pl.ds(s * RPT, RPT)],
                    agg.at[c, pl.ds(s * RPT, RPT)])

    @pl.when(s == 0)
    def _():
        pltpu.sync_copy(w_sp, wout.at[c])


def _agg_call(srcf, dstf, hsc, ndst1, z1, z2):
    return pl.kernel(
        _agg_body,
        out_type=(
            jax.ShapeDtypeStruct((NC, NP, H), _f32),
            jax.ShapeDtypeStruct((NC, NP), _f32),
        ),
        mesh=_mesh,
        scratch_types=dict(
            sidx=pltpu.VMEM((4, K), jnp.int32),
            didx=pltpu.VMEM((4, K), jnp.int32),
            stage=pltpu.VMEM((2, K, H), _f32),
            wvals=pltpu.VMEM((2, K), _f32),
            acc_sp=pltpu.VMEM_SHARED((NP, H), _f32),
            w_sp=pltpu.VMEM_SHARED((NP,), _f32),
            isem=pltpu.SemaphoreType.DMA,
            gsem=pltpu.SemaphoreType.DMA,
            wsem=pltpu.SemaphoreType.DMA,
        ),
    )(srcf, dstf, hsc, ndst1, z1, z2)


def _final_body(agg_ref, ndst_ref, nsrc_ref, wp_ref,
                b1_ref, w2_ref, b2_ref, wc_ref, bc_ref, out_ref):
    agg = agg_ref[0] + agg_ref[1]
    h1 = jnp.maximum(agg * ndst_ref[...] + b1_ref[...], 0.0)
    wsum = wp_ref[:, 0:1] + wp_ref[:, 1:2]
    real = (lax.broadcasted_iota(jnp.int32, (NP, 1), 0) < N).astype(_f32)
    coeff = wsum * nsrc_ref[...] * real
    u = lax.dot_general(coeff, h1, (((0,), (0,)), ((), ())),
                        preferred_element_type=_f32)
    hg = jnp.dot(u, w2_ref[...], preferred_element_type=_f32) * (1.0 / N)
    hg = hg + b2_ref[...]
    out_ref[...] = jnp.dot(hg, wc_ref[...], preferred_element_type=_f32) \
        + bc_ref[...]


def _final_call(agg, ndst, nsrc, wp_t, b1, w2, b2, wc, bc):
    return pl.pallas_call(
        _final_body,
        out_shape=jax.ShapeDtypeStruct((1, 10), _f32),
    )(agg, ndst, nsrc, wp_t, b1, w2, b2, wc, bc)


@jax.jit
def kernel(in_feat, edge_index, W1, b1, W2, b2, Wc, bc):
    pad = N + (jnp.arange(EP - E, dtype=jnp.int32) % (NP - N))
    srcf = jnp.concatenate([edge_index[0], pad])
    dstf = jnp.concatenate([edge_index[1], pad])
    x_pad = jnp.concatenate([in_feat, jnp.zeros((NP - N, H), _f32)])
    z1 = jnp.zeros((NP,), _f32)
    z2 = jnp.zeros((NP, H), _f32)

    degout, degin = _deg_call(srcf.reshape(NW, CHA, KA),
                              dstf.reshape(NW, CHA, KA), z1)
    hsc, nsrc, ndst = _norm_mm_call(x_pad, W1, degout.T, degin.T)
    agg, w_parts = _agg_call(srcf, dstf, hsc, ndst[:, 0], z1, z2)
    return _final_call(agg, ndst, nsrc, w_parts.T,
                       b1.reshape(1, H), W2, b2.reshape(1, H),
                       Wc.reshape(H, 10), bc.reshape(1, 10))


# speedup vs baseline: 2.7390x; 1.1631x over previous
import jax
import jax.numpy as jnp
from jax import lax
from jax.experimental import pallas as pl
from jax.experimental.pallas import tpu as pltpu
from jax.experimental.pallas import tpu_sc as plsc

N = 10000
E = 320000
H = 128
NP = 10240
NC = 2
NS = 16
NW = NC * NS
EPT = NP
EP = NW * EPT
K = 80
CH = EPT // K
KA = 128
CHA = EPT // KA
RPT = NP // NS

_mesh = plsc.VectorSubcoreMesh(core_axis_name="c", subcore_axis_name="s")
_f32 = jnp.float32


def _deg_body(srcr, dstr, z1, degout, degin,
              srcb, dstb, ones_v, go_sp, gi_sp, osem):
    c = lax.axis_index("c")
    s = lax.axis_index("s")
    wid = c * NS + s
    pltpu.sync_copy(z1.at[pl.ds(s * RPT, RPT)], go_sp.at[pl.ds(s * RPT, RPT)])
    pltpu.sync_copy(z1.at[pl.ds(s * RPT, RPT)], gi_sp.at[pl.ds(s * RPT, RPT)])
    pltpu.sync_copy(srcr.at[wid], srcb)
    pltpu.sync_copy(dstr.at[wid], dstb)
    for off in range(0, KA - 15, 16):
        ones_v[pl.ds(off, 16)] = jnp.ones((16,), _f32)
    plsc.subcore_barrier()

    def chunk(j, carry):
        pltpu.async_copy(ones_v, go_sp.at[srcb.at[j]], osem, add=True)
        pltpu.async_copy(ones_v, gi_sp.at[dstb.at[j]], osem, add=True)

        @pl.when(j >= 4)
        def _():
            pltpu.make_async_copy(ones_v, go_sp.at[srcb.at[0]], osem).wait()
            pltpu.make_async_copy(ones_v, gi_sp.at[dstb.at[0]], osem).wait()

        return carry

    lax.fori_loop(0, CHA, chunk, 0, unroll=False)
    for _ in range(4):
        pltpu.make_async_copy(ones_v, go_sp.at[srcb.at[0]], osem).wait()
        pltpu.make_async_copy(ones_v, gi_sp.at[dstb.at[0]], osem).wait()
    plsc.subcore_barrier()

    @pl.when(s == 0)
    def _():
        pltpu.sync_copy(go_sp, degout.at[c])
        pltpu.sync_copy(gi_sp, degin.at[c])


def _deg_call(srcr, dstr, z1):
    return pl.kernel(
        _deg_body,
        out_type=(
            jax.ShapeDtypeStruct((NC, NP), _f32),
            jax.ShapeDtypeStruct((NC, NP), _f32),
        ),
        mesh=_mesh,
        scratch_types=dict(
            srcb=pltpu.VMEM((CHA, KA), jnp.int32),
            dstb=pltpu.VMEM((CHA, KA), jnp.int32),
            ones_v=pltpu.VMEM((KA,), _f32),
            go_sp=pltpu.VMEM_SHARED((NP,), _f32),
            gi_sp=pltpu.VMEM_SHARED((NP,), _f32),
            osem=pltpu.SemaphoreType.DMA,
        ),
    )(srcr, dstr, z1)


def _norm_mm_body(x_ref, w1_ref, dgo_ref, dgi_ref,
                  hsc_ref, nsrc_ref, ndst_ref):
    dgo = dgo_ref[:, 0:1] + dgo_ref[:, 1:2]
    dgi = dgi_ref[:, 0:1] + dgi_ref[:, 1:2]
    nsrc = lax.rsqrt(jnp.maximum(dgo, 1.0))
    ndst = lax.rsqrt(jnp.maximum(dgi, 1.0))
    nsrc_ref[...] = nsrc
    ndst_ref[...] = ndst
    xw = jnp.dot(x_ref[...], w1_ref[...], preferred_element_type=_f32)
    hsc_ref[...] = xw * nsrc


def _norm_mm_call(x_pad, w1, dgo_t, dgi_t):
    return pl.pallas_call(
        _norm_mm_body,
        out_shape=(
            jax.ShapeDtypeStruct((NP, H), _f32),
            jax.ShapeDtypeStruct((NP, 1), _f32),
            jax.ShapeDtypeStruct((NP, 1), _f32),
        ),
    )(x_pad, w1, dgo_t, dgi_t)


def _agg_body(srcf, dstf, hsc, ndst1, z1, z2, agg, wout,
              sidx, didx, stage, wvals, acc_sp, w_sp,
              isem, gsem, wsem, ssem, vsem):
    c = lax.axis_index("c")
    s = lax.axis_index("s")
    wid = c * NS + s
    base = wid * EPT
    pltpu.sync_copy(z2.at[pl.ds(s * RPT, RPT)], acc_sp.at[pl.ds(s * RPT, RPT)])
    pltpu.sync_copy(z1.at[pl.ds(s * RPT, RPT)], w_sp.at[pl.ds(s * RPT, RPT)])
    plsc.subcore_barrier()

    def fetch_idx(j, slot):
        pltpu.async_copy(srcf.at[pl.ds(base + j * K, K)], sidx.at[slot], isem)
        pltpu.async_copy(dstf.at[pl.ds(base + j * K, K)], didx.at[slot], isem)

    def wait_idx(j, slot):
        pltpu.make_async_copy(
            srcf.at[pl.ds(base + j * K, K)], sidx.at[slot], isem).wait()
        pltpu.make_async_copy(
            dstf.at[pl.ds(base + j * K, K)], didx.at[slot], isem).wait()

    def issue_gathers(j, slot, b):
        pltpu.async_copy(hsc.at[sidx.at[slot]], stage.at[b], gsem)
        pltpu.async_copy(ndst1.at[didx.at[slot]], wvals.at[b], wsem)

    for j in range(4):
        fetch_idx(j, j)
    for j in range(2):
        wait_idx(j, j)
        issue_gathers(j, j, j)

    def chunk(j, carry):
        b3 = lax.rem(j, 3)
        b4 = lax.rem(j, 4)
        pltpu.make_async_copy(hsc.at[sidx.at[b4]], stage.at[b3], gsem).wait()
        pltpu.make_async_copy(ndst1.at[didx.at[b4]], wvals.at[b3], wsem).wait()
        pltpu.async_copy(stage.at[b3], acc_sp.at[didx.at[b4]], ssem, add=True)
        pltpu.async_copy(wvals.at[b3], w_sp.at[sidx.at[b4]], vsem, add=True)

        @pl.when(j + 4 < CH)
        def _():
            fetch_idx(j + 4, b4)

        @pl.when((j + 2 < CH) & (j >= 1))
        def _():
            slot_n = lax.rem(j + 2, 3)
            pltpu.make_async_copy(
                stage.at[slot_n], acc_sp.at[didx.at[b4]], ssem).wait()
            pltpu.make_async_copy(
                wvals.at[slot_n], w_sp.at[sidx.at[b4]], vsem).wait()

        @pl.when(j + 2 < CH)
        def _():
            wait_idx(j + 2, lax.rem(j + 2, 4))
            issue_gathers(j + 2, lax.rem(j + 2, 4), lax.rem(j + 2, 3))

        return carry

    lax.fori_loop(0, CH, chunk, 0, unroll=False)
    for _ in range(3):
        pltpu.make_async_copy(stage.at[0], acc_sp.at[didx.at[0]], ssem).wait()
        pltpu.make_async_copy(wvals.at[0], w_sp.at[sidx.at[0]], vsem).wait()
    plsc.subcore_barrier()

    pltpu.sync_copy(acc_sp.at[pl.ds(s * RPT, RPT)],
                    agg.at[c, pl.ds(s * RPT, RPT)])

    @pl.when(s == 0)
    def _():
        pltpu.sync_copy(w_sp, wout.at[c])


def _agg_call(srcf, dstf, hsc, ndst1, z1, z2):
    return pl.kernel(
        _agg_body,
        out_type=(
            jax.ShapeDtypeStruct((NC, NP, H), _f32),
            jax.ShapeDtypeStruct((NC, NP), _f32),
        ),
        mesh=_mesh,
        scratch_types=dict(
            sidx=pltpu.VMEM((4, K), jnp.int32),
            didx=pltpu.VMEM((4, K), jnp.int32),
            stage=pltpu.VMEM((3, K, H), _f32),
            wvals=pltpu.VMEM((3, K), _f32),
            acc_sp=pltpu.VMEM_SHARED((NP, H), _f32),
            w_sp=pltpu.VMEM_SHARED((NP,), _f32),
            isem=pltpu.SemaphoreType.DMA,
            gsem=pltpu.SemaphoreType.DMA,
            wsem=pltpu.SemaphoreType.DMA,
            ssem=pltpu.SemaphoreType.DMA,
            vsem=pltpu.SemaphoreType.DMA,
        ),
    )(srcf, dstf, hsc, ndst1, z1, z2)


def _final_body(agg_ref, ndst_ref, nsrc_ref, wp_ref,
                b1_ref, w2_ref, b2_ref, wc_ref, bc_ref, out_ref):
    agg = agg_ref[0] + agg_ref[1]
    h1 = jnp.maximum(agg * ndst_ref[...] + b1_ref[...], 0.0)
    wsum = wp_ref[:, 0:1] + wp_ref[:, 1:2]
    real = (lax.broadcasted_iota(jnp.int32, (NP, 1), 0) < N).astype(_f32)
    coeff = wsum * nsrc_ref[...] * real
    u = lax.dot_general(coeff, h1, (((0,), (0,)), ((), ())),
                        preferred_element_type=_f32)
    hg = jnp.dot(u, w2_ref[...], preferred_element_type=_f32) * (1.0 / N)
    hg = hg + b2_ref[...]
    out_ref[...] = jnp.dot(hg, wc_ref[...], preferred_element_type=_f32) \
        + bc_ref[...]


def _final_call(agg, ndst, nsrc, wp_t, b1, w2, b2, wc, bc):
    return pl.pallas_call(
        _final_body,
        out_shape=jax.ShapeDtypeStruct((1, 10), _f32),
    )(agg, ndst, nsrc, wp_t, b1, w2, b2, wc, bc)


@jax.jit
def kernel(in_feat, edge_index, W1, b1, W2, b2, Wc, bc):
    pad = N + (jnp.arange(EP - E, dtype=jnp.int32) % (NP - N))
    srcf = jnp.concatenate([edge_index[0], pad])
    dstf = jnp.concatenate([edge_index[1], pad])
    x_pad = jnp.concatenate([in_feat, jnp.zeros((NP - N, H), _f32)])
    z1 = jnp.zeros((NP,), _f32)
    z2 = jnp.zeros((NP, H), _f32)

    degout, degin = _deg_call(srcf.reshape(NW, CHA, KA),
                              dstf.reshape(NW, CHA, KA), z1)
    hsc, nsrc, ndst = _norm_mm_call(x_pad, W1, degout.T, degin.T)
    agg, w_parts = _agg_call(srcf, dstf, hsc, ndst[:, 0], z1, z2)
    return _final_call(agg, ndst, nsrc, w_parts.T,
                       b1.reshape(1, H), W2, b2.reshape(1, H),
                       Wc.reshape(H, 10), bc.reshape(1, 10))


# speedup vs baseline: 2.7437x; 1.0017x over previous
import jax
import jax.numpy as jnp
from jax import lax
from jax.experimental import pallas as pl
from jax.experimental.pallas import tpu as pltpu
from jax.experimental.pallas import tpu_sc as plsc

N = 10000
E = 320000
H = 128
NP = 10240
NC = 2
NS = 16
NW = NC * NS
EPT = E // NW
K = 80
CH = EPT // K
RPT = NP // NS

_mesh = plsc.VectorSubcoreMesh(core_axis_name="c", subcore_axis_name="s")
_f32 = jnp.float32


def _deg_body(srcr, dstr, z1, degout, degin,
              srcb, dstb, ones_v, go_sp, gi_sp, osem):
    c = lax.axis_index("c")
    s = lax.axis_index("s")
    wid = c * NS + s
    pltpu.sync_copy(z1.at[pl.ds(s * RPT, RPT)], go_sp.at[pl.ds(s * RPT, RPT)])
    pltpu.sync_copy(z1.at[pl.ds(s * RPT, RPT)], gi_sp.at[pl.ds(s * RPT, RPT)])
    pltpu.sync_copy(srcr.at[wid], srcb)
    pltpu.sync_copy(dstr.at[wid], dstb)
    for off in range(0, K - 15, 16):
        ones_v[pl.ds(off, 16)] = jnp.ones((16,), _f32)
    plsc.subcore_barrier()

    def chunk(j, carry):
        pltpu.async_copy(ones_v, go_sp.at[srcb.at[j]], osem, add=True)
        pltpu.async_copy(ones_v, gi_sp.at[dstb.at[j]], osem, add=True)

        @pl.when(j >= 4)
        def _():
            pltpu.make_async_copy(ones_v, go_sp.at[srcb.at[0]], osem).wait()
            pltpu.make_async_copy(ones_v, gi_sp.at[dstb.at[0]], osem).wait()

        return carry

    lax.fori_loop(0, CH, chunk, 0, unroll=False)
    for _ in range(4):
        pltpu.make_async_copy(ones_v, go_sp.at[srcb.at[0]], osem).wait()
        pltpu.make_async_copy(ones_v, gi_sp.at[dstb.at[0]], osem).wait()
    plsc.subcore_barrier()

    @pl.when(s == 0)
    def _():
        pltpu.sync_copy(go_sp, degout.at[c])
        pltpu.sync_copy(gi_sp, degin.at[c])


def _deg_call(srcr, dstr, z1):
    return pl.kernel(
        _deg_body,
        out_type=(
            jax.ShapeDtypeStruct((NC, NP), _f32),
            jax.ShapeDtypeStruct((NC, NP), _f32),
        ),
        mesh=_mesh,
        scratch_types=dict(
            srcb=pltpu.VMEM((CH, K), jnp.int32),
            dstb=pltpu.VMEM((CH, K), jnp.int32),
            ones_v=pltpu.VMEM((K,), _f32),
            go_sp=pltpu.VMEM_SHARED((NP,), _f32),
            gi_sp=pltpu.VMEM_SHARED((NP,), _f32),
            osem=pltpu.SemaphoreType.DMA,
        ),
    )(srcr, dstr, z1)


def _norm_mm_body(x_ref, w1_ref, dgo_ref, dgi_ref,
                  hsc_ref, nsrc_ref, ndst_ref):
    dgo = dgo_ref[:, 0:1] + dgo_ref[:, 1:2]
    dgi = dgi_ref[:, 0:1] + dgi_ref[:, 1:2]
    nsrc = lax.rsqrt(jnp.maximum(dgo, 1.0))
    ndst = lax.rsqrt(jnp.maximum(dgi, 1.0))
    nsrc_ref[...] = nsrc
    ndst_ref[...] = ndst
    xw = jnp.dot(x_ref[...], w1_ref[...], preferred_element_type=_f32)
    hsc_ref[0:N] = xw * nsrc[0:N]
    hsc_ref[N:NP] = jnp.zeros((NP - N, H), _f32)


def _norm_mm_call(x, w1, dgo_t, dgi_t):
    return pl.pallas_call(
        _norm_mm_body,
        out_shape=(
            jax.ShapeDtypeStruct((NP, H), _f32),
            jax.ShapeDtypeStruct((NP, 1), _f32),
            jax.ShapeDtypeStruct((NP, 1), _f32),
        ),
    )(x, w1, dgo_t, dgi_t)


def _agg_body(srcf, dstf, hsc, ndst1, z1, z2, agg, wout,
              sidx, didx, stage, wvals, acc_sp, w_sp,
              isem, gsem, wsem, ssem, vsem):
    c = lax.axis_index("c")
    s = lax.axis_index("s")
    wid = c * NS + s
    base = wid * EPT
    pltpu.sync_copy(z2.at[pl.ds(s * RPT, RPT)], acc_sp.at[pl.ds(s * RPT, RPT)])
    pltpu.sync_copy(z1.at[pl.ds(s * RPT, RPT)], w_sp.at[pl.ds(s * RPT, RPT)])
    plsc.subcore_barrier()

    def fetch_idx(j, slot):
        pltpu.async_copy(srcf.at[pl.ds(base + j * K, K)], sidx.at[slot], isem)
        pltpu.async_copy(dstf.at[pl.ds(base + j * K, K)], didx.at[slot], isem)

    def wait_idx(j, slot):
        pltpu.make_async_copy(
            srcf.at[pl.ds(base + j * K, K)], sidx.at[slot], isem).wait()
        pltpu.make_async_copy(
            dstf.at[pl.ds(base + j * K, K)], didx.at[slot], isem).wait()

    def issue_gathers(j, slot, b):
        pltpu.async_copy(hsc.at[sidx.at[slot]], stage.at[b], gsem)
        pltpu.async_copy(ndst1.at[didx.at[slot]], wvals.at[b], wsem)

    for j in range(4):
        fetch_idx(j, j)
    for j in range(2):
        wait_idx(j, j)
        issue_gathers(j, j, j)

    def chunk(j, carry):
        b3 = lax.rem(j, 3)
        b4 = lax.rem(j, 4)
        pltpu.make_async_copy(hsc.at[sidx.at[b4]], stage.at[b3], gsem).wait()
        pltpu.make_async_copy(ndst1.at[didx.at[b4]], wvals.at[b3], wsem).wait()
        pltpu.async_copy(stage.at[b3], acc_sp.at[didx.at[b4]], ssem, add=True)
        pltpu.async_copy(wvals.at[b3], w_sp.at[sidx.at[b4]], vsem, add=True)

        @pl.when(j + 4 < CH)
        def _():
            fetch_idx(j + 4, b4)

        @pl.when((j + 2 < CH) & (j >= 1))
        def _():
            slot_n = lax.rem(j + 2, 3)
            pltpu.make_async_copy(
                stage.at[slot_n], acc_sp.at[didx.at[b4]], ssem).wait()
            pltpu.make_async_copy(
                wvals.at[slot_n], w_sp.at[sidx.at[b4]], vsem).wait()

        @pl.when(j + 2 < CH)
        def _():
            wait_idx(j + 2, lax.rem(j + 2, 4))
            issue_gathers(j + 2, lax.rem(j + 2, 4), lax.rem(j + 2, 3))

        return carry

    lax.fori_loop(0, CH, chunk, 0, unroll=False)
    for _ in range(3):
        pltpu.make_async_copy(stage.at[0], acc_sp.at[didx.at[0]], ssem).wait()
        pltpu.make_async_copy(wvals.at[0], w_sp.at[sidx.at[0]], vsem).wait()
    plsc.subcore_barrier()

    pltpu.sync_copy(acc_sp.at[pl.ds(s * RPT, RPT)],
                    agg.at[c, pl.ds(s * RPT, RPT)])

    @pl.when(s == 0)
    def _():
        pltpu.sync_copy(w_sp, wout.at[c])


def _agg_call(srcf, dstf, hsc, ndst1, z1, z2):
    return pl.kernel(
        _agg_body,
        out_type=(
            jax.ShapeDtypeStruct((NC, NP, H), _f32),
            jax.ShapeDtypeStruct((NC, NP), _f32),
        ),
        mesh=_mesh,
        scratch_types=dict(
            sidx=pltpu.VMEM((4, K), jnp.int32),
            didx=pltpu.VMEM((4, K), jnp.int32),
            stage=pltpu.VMEM((3, K, H), _f32),
            wvals=pltpu.VMEM((3, K), _f32),
            acc_sp=pltpu.VMEM_SHARED((NP, H), _f32),
            w_sp=pltpu.VMEM_SHARED((NP,), _f32),
            isem=pltpu.SemaphoreType.DMA,
            gsem=pltpu.SemaphoreType.DMA,
            wsem=pltpu.SemaphoreType.DMA,
            ssem=pltpu.SemaphoreType.DMA,
            vsem=pltpu.SemaphoreType.DMA,
        ),
    )(srcf, dstf, hsc, ndst1, z1, z2)


def _final_body(agg_ref, ndst_ref, nsrc_ref, wp_ref,
                b1_ref, w2_ref, b2_ref, wc_ref, bc_ref, out_ref):
    agg = agg_ref[0] + agg_ref[1]
    h1 = jnp.maximum(agg * ndst_ref[...] + b1_ref[...], 0.0)
    wsum = wp_ref[:, 0:1] + wp_ref[:, 1:2]
    real = (lax.broadcasted_iota(jnp.int32, (NP, 1), 0) < N).astype(_f32)
    coeff = wsum * nsrc_ref[...] * real
    u = lax.dot_general(coeff, h1, (((0,), (0,)), ((), ())),
                        preferred_element_type=_f32)
    hg = jnp.dot(u, w2_ref[...], preferred_element_type=_f32) * (1.0 / N)
    hg = hg + b2_ref[...]
    out_ref[...] = jnp.dot(hg, wc_ref[...], preferred_element_type=_f32) \
        + bc_ref[...]


def _final_call(agg, ndst, nsrc, wp_t, b1, w2, b2, wc, bc):
    return pl.pallas_call(
        _final_body,
        out_shape=jax.ShapeDtypeStruct((1, 10), _f32),
    )(agg, ndst, nsrc, wp_t, b1, w2, b2, wc, bc)


@jax.jit
def kernel(in_feat, edge_index, W1, b1, W2, b2, Wc, bc):
    srcf = edge_index[0]
    dstf = edge_index[1]
    z1 = jnp.zeros((NP,), _f32)
    z2 = jnp.zeros((NP, H), _f32)

    degout, degin = _deg_call(srcf.reshape(NW, CH, K),
                              dstf.reshape(NW, CH, K), z1)
    hsc, nsrc, ndst = _norm_mm_call(in_feat, W1, degout.T, degin.T)
    agg, w_parts = _agg_call(srcf, dstf, hsc, ndst[:, 0], z1, z2)
    return _final_call(agg, ndst, nsrc, w_parts.T,
                       b1.reshape(1, H), W2, b2.reshape(1, H),
                       Wc.reshape(H, 10), bc.reshape(1, 10))


# speedup vs baseline: 3.1236x; 1.1385x over previous
import jax
import jax.numpy as jnp
from jax import lax
from jax.experimental import pallas as pl
from jax.experimental.pallas import tpu as pltpu
from jax.experimental.pallas import tpu_sc as plsc

N = 10000
E = 320000
H = 128
NP = 10240
NC = 2
NS = 16
NW = NC * NS
EPT = E // NW
K = 80
CH = EPT // K
RPT = NP // NS

_mesh = plsc.VectorSubcoreMesh(core_axis_name="c", subcore_axis_name="s")
_f32 = jnp.float32


def _deg_body(e4, z1, degout, degin,
              srcb, dstb, ones_v, go_sp, gi_sp, osem):
    c = lax.axis_index("c")
    s = lax.axis_index("s")
    wid = c * NS + s
    pltpu.sync_copy(z1.at[pl.ds(s * RPT, RPT)], go_sp.at[pl.ds(s * RPT, RPT)])
    pltpu.sync_copy(z1.at[pl.ds(s * RPT, RPT)], gi_sp.at[pl.ds(s * RPT, RPT)])
    pltpu.sync_copy(e4.at[0, wid], srcb)
    pltpu.sync_copy(e4.at[1, wid], dstb)
    for off in range(0, K - 15, 16):
        ones_v[pl.ds(off, 16)] = jnp.ones((16,), _f32)
    plsc.subcore_barrier()

    def chunk(j, carry):
        pltpu.async_copy(ones_v, go_sp.at[srcb.at[j]], osem, add=True)
        pltpu.async_copy(ones_v, gi_sp.at[dstb.at[j]], osem, add=True)

        @pl.when(j >= 4)
        def _():
            pltpu.make_async_copy(ones_v, go_sp.at[srcb.at[0]], osem).wait()
            pltpu.make_async_copy(ones_v, gi_sp.at[dstb.at[0]], osem).wait()

        return carry

    lax.fori_loop(0, CH, chunk, 0, unroll=False)
    for _ in range(4):
        pltpu.make_async_copy(ones_v, go_sp.at[srcb.at[0]], osem).wait()
        pltpu.make_async_copy(ones_v, gi_sp.at[dstb.at[0]], osem).wait()
    plsc.subcore_barrier()

    @pl.when(s == 0)
    def _():
        pltpu.sync_copy(go_sp, degout.at[c])
        pltpu.sync_copy(gi_sp, degin.at[c])


def _deg_call(e4, z1):
    return pl.kernel(
        _deg_body,
        out_type=(
            jax.ShapeDtypeStruct((NC, NP), _f32),
            jax.ShapeDtypeStruct((NC, NP), _f32),
        ),
        mesh=_mesh,
        scratch_types=dict(
            srcb=pltpu.VMEM((CH, K), jnp.int32),
            dstb=pltpu.VMEM((CH, K), jnp.int32),
            ones_v=pltpu.VMEM((K,), _f32),
            go_sp=pltpu.VMEM_SHARED((NP,), _f32),
            gi_sp=pltpu.VMEM_SHARED((NP,), _f32),
            osem=pltpu.SemaphoreType.DMA,
        ),
    )(e4, z1)


def _norm_mm_body(x_ref, w1_ref, dgo_ref, dgi_ref,
                  hsc_ref, nsrc_ref, ndst_ref):
    ones21 = jnp.ones((2, 1), _f32)
    dgo = lax.dot_general(dgo_ref[...], ones21, (((0,), (0,)), ((), ())),
                          preferred_element_type=_f32)
    dgi = lax.dot_general(dgi_ref[...], ones21, (((0,), (0,)), ((), ())),
                          preferred_element_type=_f32)
    nsrc = lax.rsqrt(jnp.maximum(dgo, 1.0))
    ndst = lax.rsqrt(jnp.maximum(dgi, 1.0))
    nsrc_ref[...] = nsrc
    ndst_ref[...] = ndst
    xw = jnp.dot(x_ref[...], w1_ref[...], preferred_element_type=_f32)
    hsc_ref[0:N] = xw * nsrc[0:N]
    hsc_ref[N:NP] = jnp.zeros((NP - N, H), _f32)


def _norm_mm_call(x, w1, dgo, dgi):
    return pl.pallas_call(
        _norm_mm_body,
        out_shape=(
            jax.ShapeDtypeStruct((NP, H), _f32),
            jax.ShapeDtypeStruct((NP, 1), _f32),
            jax.ShapeDtypeStruct((NP, 1), _f32),
        ),
    )(x, w1, dgo, dgi)


def _agg_body(eflat, hsc, ndst1, z1, z2, agg, wout,
              sidx, didx, stage, wvals, acc_sp, w_sp,
              isem, gsem, wsem, ssem, vsem):
    c = lax.axis_index("c")
    s = lax.axis_index("s")
    wid = c * NS + s
    base = wid * EPT
    pltpu.sync_copy(z2.at[pl.ds(s * RPT, RPT)], acc_sp.at[pl.ds(s * RPT, RPT)])
    pltpu.sync_copy(z1.at[pl.ds(s * RPT, RPT)], w_sp.at[pl.ds(s * RPT, RPT)])
    plsc.subcore_barrier()

    def fetch_idx(j, slot):
        pltpu.async_copy(
            eflat.at[pl.ds(base + j * K, K)], sidx.at[slot], isem)
        pltpu.async_copy(
            eflat.at[pl.ds(E + base + j * K, K)], didx.at[slot], isem)

    def wait_idx(j, slot):
        pltpu.make_async_copy(
            eflat.at[pl.ds(base + j * K, K)], sidx.at[slot], isem).wait()
        pltpu.make_async_copy(
            eflat.at[pl.ds(E + base + j * K, K)], didx.at[slot], isem).wait()

    def issue_gathers(j, slot, b):
        pltpu.async_copy(hsc.at[sidx.at[slot]], stage.at[b], gsem)
        pltpu.async_copy(ndst1.at[didx.at[slot]], wvals.at[b], wsem)

    for j in range(4):
        fetch_idx(j, j)
    for j in range(2):
        wait_idx(j, j)
        issue_gathers(j, j, j)

    def chunk(j, carry):
        b3 = lax.rem(j, 3)
        b4 = lax.rem(j, 4)
        pltpu.make_async_copy(hsc.at[sidx.at[b4]], stage.at[b3], gsem).wait()
        pltpu.make_async_copy(ndst1.at[didx.at[b4]], wvals.at[b3], wsem).wait()
        pltpu.async_copy(stage.at[b3], acc_sp.at[didx.at[b4]], ssem, add=True)
        pltpu.async_copy(wvals.at[b3], w_sp.at[sidx.at[b4]], vsem, add=True)

        @pl.when(j + 4 < CH)
        def _():
            fetch_idx(j + 4, b4)

        @pl.when((j + 2 < CH) & (j >= 1))
        def _():
            slot_n = lax.rem(j + 2, 3)
            pltpu.make_async_copy(
                stage.at[slot_n], acc_sp.at[didx.at[b4]], ssem).wait()
            pltpu.make_async_copy(
                wvals.at[slot_n], w_sp.at[sidx.at[b4]], vsem).wait()

        @pl.when(j + 2 < CH)
        def _():
            wait_idx(j + 2, lax.rem(j + 2, 4))
            issue_gathers(j + 2, lax.rem(j + 2, 4), lax.rem(j + 2, 3))

        return carry

    lax.fori_loop(0, CH, chunk, 0, unroll=False)
    for _ in range(3):
        pltpu.make_async_copy(stage.at[0], acc_sp.at[didx.at[0]], ssem).wait()
        pltpu.make_async_copy(wvals.at[0], w_sp.at[sidx.at[0]], vsem).wait()
    plsc.subcore_barrier()

    pltpu.sync_copy(acc_sp.at[pl.ds(s * RPT, RPT)],
                    agg.at[c, pl.ds(s * RPT, RPT)])

    @pl.when(s == 0)
    def _():
        pltpu.sync_copy(w_sp, wout.at[c])


def _agg_call(eflat, hsc, ndst1, z1, z2):
    return pl.kernel(
        _agg_body,
        out_type=(
            jax.ShapeDtypeStruct((NC, NP, H), _f32),
            jax.ShapeDtypeStruct((NC, NP), _f32),
        ),
        mesh=_mesh,
        scratch_types=dict(
            sidx=pltpu.VMEM((4, K), jnp.int32),
            didx=pltpu.VMEM((4, K), jnp.int32),
            stage=pltpu.VMEM((3, K, H), _f32),
            wvals=pltpu.VMEM((3, K), _f32),
            acc_sp=pltpu.VMEM_SHARED((NP, H), _f32),
            w_sp=pltpu.VMEM_SHARED((NP,), _f32),
            isem=pltpu.SemaphoreType.DMA,
            gsem=pltpu.SemaphoreType.DMA,
            wsem=pltpu.SemaphoreType.DMA,
            ssem=pltpu.SemaphoreType.DMA,
            vsem=pltpu.SemaphoreType.DMA,
        ),
    )(eflat, hsc, ndst1, z1, z2)


def _final_body(agg_ref, ndst_ref, nsrc_ref, wp_ref,
                b1_ref, w2_ref, b2_ref, wc_ref, bc_ref, out_ref):
    agg = agg_ref[0] + agg_ref[1]
    h1 = jnp.maximum(agg * ndst_ref[...] + b1_ref[...], 0.0)
    ones21 = jnp.ones((2, 1), _f32)
    wsum = lax.dot_general(wp_ref[...], ones21, (((0,), (0,)), ((), ())),
                           preferred_element_type=_f32)
    real = (lax.broadcasted_iota(jnp.int32, (NP, 1), 0) < N).astype(_f32)
    coeff = wsum * nsrc_ref[...] * real
    u = lax.dot_general(coeff, h1, (((0,), (0,)), ((), ())),
                        preferred_element_type=_f32)
    hg = jnp.dot(u, w2_ref[...], preferred_element_type=_f32) * (1.0 / N)
    hg = hg + b2_ref[...]
    out_ref[...] = jnp.dot(hg, wc_ref[...], preferred_element_type=_f32) \
        + bc_ref[...]


def _final_call(agg, ndst, nsrc, wp, b1, w2, b2, wc, bc):
    return pl.pallas_call(
        _final_body,
        out_shape=jax.ShapeDtypeStruct((1, 10), _f32),
    )(agg, ndst, nsrc, wp, b1, w2, b2, wc, bc)


@jax.jit
def kernel(in_feat, edge_index, W1, b1, W2, b2, Wc, bc):
    e4 = edge_index.reshape(2, NW, CH, K)
    eflat = edge_index.reshape(2 * E)
    z1 = jnp.zeros((NP,), _f32)
    z2 = jnp.zeros((NP, H), _f32)

    degout, degin = _deg_call(e4, z1)
    hsc, nsrc, ndst = _norm_mm_call(in_feat, W1, degout, degin)
    agg, w_parts = _agg_call(eflat, hsc, ndst.reshape(NP), z1, z2)
    return _final_call(agg, ndst, nsrc, w_parts,
                       b1.reshape(1, H), W2, b2.reshape(1, H),
                       Wc.reshape(H, 10), bc.reshape(1, 10))


# speedup vs baseline: 3.1583x; 1.0111x over previous
import jax
import jax.numpy as jnp
from jax import lax
from jax.experimental import pallas as pl
from jax.experimental.pallas import tpu as pltpu
from jax.experimental.pallas import tpu_sc as plsc

N = 10000
E = 320000
H = 128
NP = 10240
NC = 2
NS = 16
NW = NC * NS
EPT = E // NW
K = 80
CH = EPT // K
RPT = NP // NS

_mesh = plsc.VectorSubcoreMesh(core_axis_name="c", subcore_axis_name="s")
_f32 = jnp.float32


def _deg_body(eflat, z1, degout, degin,
              sidx, didx, ones_v, go_sp, gi_sp, isem, osem):
    c = lax.axis_index("c")
    s = lax.axis_index("s")
    wid = c * NS + s
    base = wid * EPT
    pltpu.sync_copy(z1.at[pl.ds(s * RPT, RPT)], go_sp.at[pl.ds(s * RPT, RPT)])
    pltpu.sync_copy(z1.at[pl.ds(s * RPT, RPT)], gi_sp.at[pl.ds(s * RPT, RPT)])
    for off in range(0, K - 15, 16):
        ones_v[pl.ds(off, 16)] = jnp.ones((16,), _f32)

    def fetch_idx(j, slot):
        pltpu.async_copy(
            eflat.at[pl.ds(base + j * K, K)], sidx.at[slot], isem)
        pltpu.async_copy(
            eflat.at[pl.ds(E + base + j * K, K)], didx.at[slot], isem)

    def wait_idx(j, slot):
        pltpu.make_async_copy(
            eflat.at[pl.ds(base + j * K, K)], sidx.at[slot], isem).wait()
        pltpu.make_async_copy(
            eflat.at[pl.ds(E + base + j * K, K)], didx.at[slot], isem).wait()

    for j in range(4):
        fetch_idx(j, j)
    plsc.subcore_barrier()

    def chunk(j, carry):
        b8 = lax.rem(j, 8)

        @pl.when(j >= 4)
        def _():
            pltpu.make_async_copy(ones_v, go_sp.at[sidx.at[0]], osem).wait()
            pltpu.make_async_copy(ones_v, gi_sp.at[didx.at[0]], osem).wait()

        @pl.when(j + 4 < CH)
        def _():
            fetch_idx(j + 4, lax.rem(j + 4, 8))

        wait_idx(j, b8)
        pltpu.async_copy(ones_v, go_sp.at[sidx.at[b8]], osem, add=True)
        pltpu.async_copy(ones_v, gi_sp.at[didx.at[b8]], osem, add=True)
        return carry

    lax.fori_loop(0, CH, chunk, 0, unroll=False)
    for _ in range(4):
        pltpu.make_async_copy(ones_v, go_sp.at[sidx.at[0]], osem).wait()
        pltpu.make_async_copy(ones_v, gi_sp.at[didx.at[0]], osem).wait()
    plsc.subcore_barrier()

    @pl.when(s == 0)
    def _():
        pltpu.sync_copy(go_sp, degout.at[c])
        pltpu.sync_copy(gi_sp, degin.at[c])


def _deg_call(eflat, z1):
    return pl.kernel(
        _deg_body,
        out_type=(
            jax.ShapeDtypeStruct((NC, NP), _f32),
            jax.ShapeDtypeStruct((NC, NP), _f32),
        ),
        mesh=_mesh,
        scratch_types=dict(
            sidx=pltpu.VMEM((8, K), jnp.int32),
            didx=pltpu.VMEM((8, K), jnp.int32),
            ones_v=pltpu.VMEM((K,), _f32),
            go_sp=pltpu.VMEM_SHARED((NP,), _f32),
            gi_sp=pltpu.VMEM_SHARED((NP,), _f32),
            isem=pltpu.SemaphoreType.DMA,
            osem=pltpu.SemaphoreType.DMA,
        ),
    )(eflat, z1)


def _norm_mm_body(x_ref, w1_ref, dgo_ref, dgi_ref,
                  hsc_ref, nsrc_ref, ndst_ref, ndst1d_ref):
    ones21 = jnp.ones((2, 1), _f32)
    dgo = lax.dot_general(dgo_ref[...], ones21, (((0,), (0,)), ((), ())),
                          preferred_element_type=_f32)
    dgi = lax.dot_general(dgi_ref[...], ones21, (((0,), (0,)), ((), ())),
                          preferred_element_type=_f32)
    nsrc = lax.rsqrt(jnp.maximum(dgo, 1.0))
    ndst = lax.rsqrt(jnp.maximum(dgi, 1.0))
    nsrc_ref[...] = nsrc
    ndst_ref[...] = ndst
    dgi1 = dgi_ref[0] + dgi_ref[1]
    ndst1d_ref[...] = lax.rsqrt(jnp.maximum(dgi1, 1.0))
    xw = jnp.dot(x_ref[...], w1_ref[...], preferred_element_type=_f32)
    hsc_ref[0:N] = xw * nsrc[0:N]
    hsc_ref[N:NP] = jnp.zeros((NP - N, H), _f32)


def _norm_mm_call(x, w1, dgo, dgi):
    return pl.pallas_call(
        _norm_mm_body,
        out_shape=(
            jax.ShapeDtypeStruct((NP, H), _f32),
            jax.ShapeDtypeStruct((NP, 1), _f32),
            jax.ShapeDtypeStruct((NP, 1), _f32),
            jax.ShapeDtypeStruct((NP,), _f32),
        ),
    )(x, w1, dgo, dgi)


def _agg_body(eflat, hsc, ndst1, agg, wout,
              sidx, didx, stage, wvals, acc_sp, w_sp,
              isem, gsem, wsem, ssem, vsem):
    c = lax.axis_index("c")
    s = lax.axis_index("s")
    wid = c * NS + s
    base = wid * EPT

    def zrow(i, carry):
        for kk in range(H // 16):
            stage[0, i, pl.ds(kk * 16, 16)] = jnp.zeros((16,), _f32)
        return carry

    lax.fori_loop(0, K, zrow, 0, unroll=False)
    for off in range(0, K - 15, 16):
        wvals[0, pl.ds(off, 16)] = jnp.zeros((16,), _f32)
    for i in range(RPT // K):
        pltpu.async_copy(
            stage.at[0], acc_sp.at[pl.ds(s * RPT + i * K, K)], ssem)
        pltpu.async_copy(
            wvals.at[0], w_sp.at[pl.ds(s * RPT + i * K, K)], vsem)

    def fetch_idx(j, slot):
        pltpu.async_copy(
            eflat.at[pl.ds(base + j * K, K)], sidx.at[slot], isem)
        pltpu.async_copy(
            eflat.at[pl.ds(E + base + j * K, K)], didx.at[slot], isem)

    def wait_idx(j, slot):
        pltpu.make_async_copy(
            eflat.at[pl.ds(base + j * K, K)], sidx.at[slot], isem).wait()
        pltpu.make_async_copy(
            eflat.at[pl.ds(E + base + j * K, K)], didx.at[slot], isem).wait()

    def issue_gathers(j, slot, b):
        pltpu.async_copy(hsc.at[sidx.at[slot]], stage.at[b], gsem)
        pltpu.async_copy(ndst1.at[didx.at[slot]], wvals.at[b], wsem)

    for j in range(4):
        fetch_idx(j, j)
    for i in range(RPT // K):
        pltpu.make_async_copy(
            stage.at[0], acc_sp.at[pl.ds(s * RPT + i * K, K)], ssem).wait()
        pltpu.make_async_copy(
            wvals.at[0], w_sp.at[pl.ds(s * RPT + i * K, K)], vsem).wait()
    plsc.subcore_barrier()
    for j in range(2):
        wait_idx(j, j)
        issue_gathers(j, j, j)

    def chunk(j, carry):
        b3 = lax.rem(j, 3)
        b8 = lax.rem(j, 8)
        pltpu.make_async_copy(hsc.at[sidx.at[b8]], stage.at[b3], gsem).wait()
        pltpu.make_async_copy(ndst1.at[didx.at[b8]], wvals.at[b3], wsem).wait()
        pltpu.async_copy(stage.at[b3], acc_sp.at[didx.at[b8]], ssem, add=True)
        pltpu.async_copy(wvals.at[b3], w_sp.at[sidx.at[b8]], vsem, add=True)

        @pl.when(j + 4 < CH)
        def _():
            fetch_idx(j + 4, lax.rem(j + 4, 8))

        @pl.when((j + 2 < CH) & (j >= 1))
        def _():
            slot_n = lax.rem(j + 2, 3)
            pltpu.make_async_copy(
                stage.at[slot_n], acc_sp.at[didx.at[b8]], ssem).wait()
            pltpu.make_async_copy(
                wvals.at[slot_n], w_sp.at[sidx.at[b8]], vsem).wait()

        @pl.when(j + 2 < CH)
        def _():
            wait_idx(j + 2, lax.rem(j + 2, 8))
            issue_gathers(j + 2, lax.rem(j + 2, 8), lax.rem(j + 2, 3))

        return carry

    lax.fori_loop(0, CH, chunk, 0, unroll=False)
    for _ in range(3):
        pltpu.make_async_copy(stage.at[0], acc_sp.at[didx.at[0]], ssem).wait()
        pltpu.make_async_copy(wvals.at[0], w_sp.at[sidx.at[0]], vsem).wait()
    plsc.subcore_barrier()

    pltpu.sync_copy(acc_sp.at[pl.ds(s * RPT, RPT)],
                    agg.at[c, pl.ds(s * RPT, RPT)])

    @pl.when(s == 0)
    def _():
        pltpu.sync_copy(w_sp, wout.at[c])


def _agg_call(eflat, hsc, ndst1):
    return pl.kernel(
        _agg_body,
        out_type=(
            jax.ShapeDtypeStruct((NC, NP, H), _f32),
            jax.ShapeDtypeStruct((NC, NP), _f32),
        ),
        mesh=_mesh,
        scratch_types=dict(
            sidx=pltpu.VMEM((8, K), jnp.int32),
            didx=pltpu.VMEM((8, K), jnp.int32),
            stage=pltpu.VMEM((3, K, H), _f32),
            wvals=pltpu.VMEM((3, K), _f32),
            acc_sp=pltpu.VMEM_SHARED((NP, H), _f32),
            w_sp=pltpu.VMEM_SHARED((NP,), _f32),
            isem=pltpu.SemaphoreType.DMA,
            gsem=pltpu.SemaphoreType.DMA,
            wsem=pltpu.SemaphoreType.DMA,
            ssem=pltpu.SemaphoreType.DMA,
            vsem=pltpu.SemaphoreType.DMA,
        ),
    )(eflat, hsc, ndst1)


def _final_body(agg_ref, ndst_ref, nsrc_ref, wp_ref,
                b1_ref, w2_ref, b2_ref, wc_ref, bc_ref, out_ref):
    agg = agg_ref[0] + agg_ref[1]
    h1 = jnp.maximum(agg * ndst_ref[...] + b1_ref[...], 0.0)
    ones21 = jnp.ones((2, 1), _f32)
    wsum = lax.dot_general(wp_ref[...], ones21, (((0,), (0,)), ((), ())),
                           preferred_element_type=_f32)
    real = (lax.broadcasted_iota(jnp.int32, (NP, 1), 0) < N).astype(_f32)
    coeff = wsum * nsrc_ref[...] * real
    u = lax.dot_general(coeff, h1, (((0,), (0,)), ((), ())),
                        preferred_element_type=_f32)
    hg = jnp.dot(u, w2_ref[...], preferred_element_type=_f32) * (1.0 / N)
    hg = hg + b2_ref[...]
    out_ref[...] = jnp.dot(hg, wc_ref[...], preferred_element_type=_f32) \
        + bc_ref[...]


def _final_call(agg, ndst, nsrc, wp, b1, w2, b2, wc, bc):
    return pl.pallas_call(
        _final_body,
        out_shape=jax.ShapeDtypeStruct((1, 10), _f32),
    )(agg, ndst, nsrc, wp, b1, w2, b2, wc, bc)


@jax.jit
def kernel(in_feat, edge_index, W1, b1, W2, b2, Wc, bc):
    eflat = edge_index.reshape(2 * E)
    z1 = jnp.zeros((NP,), _f32)

    degout, degin = _deg_call(eflat, z1)
    hsc, nsrc, ndst, ndst1d = _norm_mm_call(in_feat, W1, degout, degin)
    agg, w_parts = _agg_call(eflat, hsc, ndst1d)
    return _final_call(agg, ndst, nsrc, w_parts,
                       b1.reshape(1, H), W2, b2.reshape(1, H),
                       Wc.reshape(H, 10), bc.reshape(1, 10))


# speedup vs baseline: 3.2703x; 1.0354x over previous
import jax
import jax.numpy as jnp
from jax import lax
from jax.experimental import pallas as pl
from jax.experimental.pallas import tpu as pltpu
from jax.experimental.pallas import tpu_sc as plsc

N = 10000
E = 320000
H = 128
NP = 10240
NC = 2
NS = 16
NW = NC * NS
EPT = E // NW
K = 80
CH = EPT // K
RPT = NP // NS

_mesh = plsc.VectorSubcoreMesh(core_axis_name="c", subcore_axis_name="s")
_f32 = jnp.float32


def _deg_body(eflat, z1, degout, degin,
              sidx, didx, ones_v, go_sp, gi_sp, isem, osem):
    c = lax.axis_index("c")
    s = lax.axis_index("s")
    wid = c * NS + s
    base = wid * EPT
    pltpu.sync_copy(z1.at[pl.ds(s * RPT, RPT)], go_sp.at[pl.ds(s * RPT, RPT)])
    pltpu.sync_copy(z1.at[pl.ds(s * RPT, RPT)], gi_sp.at[pl.ds(s * RPT, RPT)])
    for off in range(0, K - 15, 16):
        ones_v[pl.ds(off, 16)] = jnp.ones((16,), _f32)

    def fetch_idx(j, slot):
        pltpu.async_copy(
            eflat.at[pl.ds(base + j * K, K)], sidx.at[slot], isem)
        pltpu.async_copy(
            eflat.at[pl.ds(E + base + j * K, K)], didx.at[slot], isem)

    def wait_idx(j, slot):
        pltpu.make_async_copy(
            eflat.at[pl.ds(base + j * K, K)], sidx.at[slot], isem).wait()
        pltpu.make_async_copy(
            eflat.at[pl.ds(E + base + j * K, K)], didx.at[slot], isem).wait()

    for j in range(8):
        fetch_idx(j, j)
    plsc.subcore_barrier()

    def chunk(j, carry):
        b16 = lax.rem(j, 16)

        @pl.when(j >= 8)
        def _():
            pltpu.make_async_copy(ones_v, go_sp.at[sidx.at[0]], osem).wait()
            pltpu.make_async_copy(ones_v, gi_sp.at[didx.at[0]], osem).wait()

        @pl.when(j + 8 < CH)
        def _():
            fetch_idx(j + 8, lax.rem(j + 8, 16))

        wait_idx(j, b16)
        pltpu.async_copy(ones_v, go_sp.at[sidx.at[b16]], osem, add=True)
        pltpu.async_copy(ones_v, gi_sp.at[didx.at[b16]], osem, add=True)
        return carry

    lax.fori_loop(0, CH, chunk, 0, unroll=False)
    for _ in range(8):
        pltpu.make_async_copy(ones_v, go_sp.at[sidx.at[0]], osem).wait()
        pltpu.make_async_copy(ones_v, gi_sp.at[didx.at[0]], osem).wait()
    plsc.subcore_barrier()

    @pl.when(s == 0)
    def _():
        pltpu.sync_copy(go_sp, degout.at[c])
        pltpu.sync_copy(gi_sp, degin.at[c])


def _deg_call(eflat, z1):
    return pl.kernel(
        _deg_body,
        out_type=(
            jax.ShapeDtypeStruct((NC, NP), _f32),
            jax.ShapeDtypeStruct((NC, NP), _f32),
        ),
        mesh=_mesh,
        scratch_types=dict(
            sidx=pltpu.VMEM((16, K), jnp.int32),
            didx=pltpu.VMEM((16, K), jnp.int32),
            ones_v=pltpu.VMEM((K,), _f32),
            go_sp=pltpu.VMEM_SHARED((NP,), _f32),
            gi_sp=pltpu.VMEM_SHARED((NP,), _f32),
            isem=pltpu.SemaphoreType.DMA,
            osem=pltpu.SemaphoreType.DMA,
        ),
    )(eflat, z1)


def _norm_mm_body(x_ref, w1_ref, dgo_ref, dgi_ref,
                  hsc_ref, nsrc_ref, ndst_ref, ndst1d_ref):
    ones21 = jnp.ones((2, 1), _f32)
    dgo = lax.dot_general(dgo_ref[...], ones21, (((0,), (0,)), ((), ())),
                          preferred_element_type=_f32)
    dgi = lax.dot_general(dgi_ref[...], ones21, (((0,), (0,)), ((), ())),
                          preferred_element_type=_f32)
    nsrc = lax.rsqrt(jnp.maximum(dgo, 1.0))
    ndst = lax.rsqrt(jnp.maximum(dgi, 1.0))
    nsrc_ref[...] = nsrc
    ndst_ref[...] = ndst
    dgi1 = dgi_ref[0] + dgi_ref[1]
    ndst1d_ref[...] = lax.rsqrt(jnp.maximum(dgi1, 1.0))
    xw = jnp.dot(x_ref[...], w1_ref[...], preferred_element_type=_f32)
    hsc_ref[0:N] = xw * nsrc[0:N]
    hsc_ref[N:NP] = jnp.zeros((NP - N, H), _f32)


def _norm_mm_call(x, w1, dgo, dgi):
    return pl.pallas_call(
        _norm_mm_body,
        out_shape=(
            jax.ShapeDtypeStruct((NP, H), _f32),
            jax.ShapeDtypeStruct((NP, 1), _f32),
            jax.ShapeDtypeStruct((NP, 1), _f32),
            jax.ShapeDtypeStruct((NP,), _f32),
        ),
    )(x, w1, dgo, dgi)


def _agg_body(eflat, hsc, ndst1, agg, wout,
              sidx, didx, stage, wvals, acc_sp, w_sp,
              isem, gsem, wsem, ssem, vsem):
    c = lax.axis_index("c")
    s = lax.axis_index("s")
    wid = c * NS + s
    base = wid * EPT

    def zrow(i, carry):
        for kk in range(H // 16):
            stage[0, i, pl.ds(kk * 16, 16)] = jnp.zeros((16,), _f32)
        return carry

    lax.fori_loop(0, K, zrow, 0, unroll=False)
    for off in range(0, K - 15, 16):
        wvals[0, pl.ds(off, 16)] = jnp.zeros((16,), _f32)
    for i in range(RPT // K):
        pltpu.async_copy(
            stage.at[0], acc_sp.at[pl.ds(s * RPT + i * K, K)], ssem)
        pltpu.async_copy(
            wvals.at[0], w_sp.at[pl.ds(s * RPT + i * K, K)], vsem)

    def fetch_idx(j, slot):
        pltpu.async_copy(
            eflat.at[pl.ds(base + j * K, K)], sidx.at[slot], isem)
        pltpu.async_copy(
            eflat.at[pl.ds(E + base + j * K, K)], didx.at[slot], isem)

    def wait_idx(j, slot):
        pltpu.make_async_copy(
            eflat.at[pl.ds(base + j * K, K)], sidx.at[slot], isem).wait()
        pltpu.make_async_copy(
            eflat.at[pl.ds(E + base + j * K, K)], didx.at[slot], isem).wait()

    def issue_gathers(j, slot, b):
        pltpu.async_copy(hsc.at[sidx.at[slot]], stage.at[b], gsem)
        pltpu.async_copy(ndst1.at[didx.at[slot]], wvals.at[b], wsem)

    for j in range(4):
        fetch_idx(j, j)
    for i in range(RPT // K):
        pltpu.make_async_copy(
            stage.at[0], acc_sp.at[pl.ds(s * RPT + i * K, K)], ssem).wait()
        pltpu.make_async_copy(
            wvals.at[0], w_sp.at[pl.ds(s * RPT + i * K, K)], vsem).wait()
    plsc.subcore_barrier()
    for j in range(2):
        wait_idx(j, j)
        issue_gathers(j, j, j)

    def chunk(j, carry):
        b4s = lax.rem(j, 4)
        b8 = lax.rem(j, 8)
        pltpu.make_async_copy(hsc.at[sidx.at[b8]], stage.at[b4s], gsem).wait()
        pltpu.make_async_copy(
            ndst1.at[didx.at[b8]], wvals.at[b4s], wsem).wait()
        pltpu.async_copy(stage.at[b4s], acc_sp.at[didx.at[b8]], ssem,
                         add=True)
        pltpu.async_copy(wvals.at[b4s], w_sp.at[sidx.at[b8]], vsem, add=True)

        @pl.when(j + 4 < CH)
        def _():
            fetch_idx(j + 4, lax.rem(j + 4, 8))

        @pl.when((j + 2 < CH) & (j >= 2))
        def _():
            slot_n = lax.rem(j + 2, 4)
            pltpu.make_async_copy(
                stage.at[slot_n], acc_sp.at[didx.at[b8]], ssem).wait()
            pltpu.make_async_copy(
                wvals.at[slot_n], w_sp.at[sidx.at[b8]], vsem).wait()

        @pl.when(j + 2 < CH)
        def _():
            wait_idx(j + 2, lax.rem(j + 2, 8))
            issue_gathers(j + 2, lax.rem(j + 2, 8), lax.rem(j + 2, 4))

        return carry

    lax.fori_loop(0, CH, chunk, 0, unroll=False)
    for _ in range(4):
        pltpu.make_async_copy(stage.at[0], acc_sp.at[didx.at[0]], ssem).wait()
        pltpu.make_async_copy(wvals.at[0], w_sp.at[sidx.at[0]], vsem).wait()
    plsc.subcore_barrier()

    pltpu.sync_copy(acc_sp.at[pl.ds(s * RPT, RPT)],
                    agg.at[c, pl.ds(s * RPT, RPT)])

    @pl.when(s == 0)
    def _():
        pltpu.sync_copy(w_sp, wout.at[c])


def _agg_call(eflat, hsc, ndst1):
    return pl.kernel(
        _agg_body,
        out_type=(
            jax.ShapeDtypeStruct((NC, NP, H), _f32),
            jax.ShapeDtypeStruct((NC, NP), _f32),
        ),
        mesh=_mesh,
        scratch_types=dict(
            sidx=pltpu.VMEM((8, K), jnp.int32),
            didx=pltpu.VMEM((8, K), jnp.int32),
            stage=pltpu.VMEM((4, K, H), _f32),
            wvals=pltpu.VMEM((4, K), _f32),
            acc_sp=pltpu.VMEM_SHARED((NP, H), _f32),
            w_sp=pltpu.VMEM_SHARED((NP,), _f32),
            isem=pltpu.SemaphoreType.DMA,
            gsem=pltpu.SemaphoreType.DMA,
            wsem=pltpu.SemaphoreType.DMA,
            ssem=pltpu.SemaphoreType.DMA,
            vsem=pltpu.SemaphoreType.DMA,
        ),
    )(eflat, hsc, ndst1)


def _final_body(agg_ref, ndst_ref, nsrc_ref, wp_ref,
                b1_ref, w2_ref, b2_ref, wc_ref, bc_ref, out_ref):
    agg = agg_ref[0] + agg_ref[1]
    h1 = jnp.maximum(agg * ndst_ref[...] + b1_ref[...], 0.0)
    ones21 = jnp.ones((2, 1), _f32)
    wsum = lax.dot_general(wp_ref[...], ones21, (((0,), (0,)), ((), ())),
                           preferred_element_type=_f32)
    real = (lax.broadcasted_iota(jnp.int32, (NP, 1), 0) < N).astype(_f32)
    coeff = wsum * nsrc_ref[...] * real
    u = lax.dot_general(coeff, h1, (((0,), (0,)), ((), ())),
                        preferred_element_type=_f32)
    hg = jnp.dot(u, w2_ref[...], preferred_element_type=_f32) * (1.0 / N)
    hg = hg + b2_ref[...]
    out_ref[...] = jnp.dot(hg, wc_ref[...], preferred_element_type=_f32) \
        + bc_ref[...]


def _final_call(agg, ndst, nsrc, wp, b1, w2, b2, wc, bc):
    return pl.pallas_call(
        _final_body,
        out_shape=jax.ShapeDtypeStruct((1, 10), _f32),
    )(agg, ndst, nsrc, wp, b1, w2, b2, wc, bc)


@jax.jit
def kernel(in_feat, edge_index, W1, b1, W2, b2, Wc, bc):
    eflat = edge_index.reshape(2 * E)
    z1 = jnp.zeros((NP,), _f32)

    degout, degin = _deg_call(eflat, z1)
    hsc, nsrc, ndst, ndst1d = _norm_mm_call(in_feat, W1, degout, degin)
    agg, w_parts = _agg_call(eflat, hsc, ndst1d)
    return _final_call(agg, ndst, nsrc, w_parts,
                       b1.reshape(1, H), W2, b2.reshape(1, H),
                       Wc.reshape(H, 10), bc.reshape(1, 10))
